# baseline (speedup 1.0000x reference)
import functools

import jax
import jax.numpy as jnp
from jax import lax
from jax.experimental import pallas as pl
from jax.experimental.pallas import tpu as pltpu
from jax.experimental.pallas import tpu_sc as plsc

N = 16384
K = 100
KSEL = K + 1
PCT_IDX = 16
BQ = 128
NW = 32
H = 1
NH = N // H
RPW = NH // NW
CAP = 256
OUTW = 128
MININT = -2147483648


def _phase1_body(xy_ref, xyt_ref, d2_ref, thr_ref):
    q = xy_ref[...]
    kxy = xyt_ref[...]
    qx = q[:, 0:1]
    qy = q[:, 1:2]
    kx = kxy[0:1, :]
    ky = kxy[1:2, :]
    qsq = qx * qx + qy * qy
    ksq = kx * kx + ky * ky
    dot = lax.dot_general(q, kxy, (((1,), (0,)), ((), ())),
                          preferred_element_type=jnp.float32)
    d2 = qsq - 2.0 * dot + ksq
    d2_ref[...] = d2

    b = lax.bitcast_convert_type(d2, jnp.int32)
    s = jnp.where(b < 0, MININT - b, b)

    def step(i, low):
        bit = lax.shift_left(jnp.int32(1), jnp.int32(31) - i)
        cand = low + bit
        cnt = jnp.sum((s < cand).astype(jnp.float32), axis=1, keepdims=True)
        return jnp.where(cnt < float(KSEL), cand, low)

    low = lax.fori_loop(0, 11, step, jnp.full((BQ, 1), MININT, jnp.int32))
    t_s = low + jnp.int32(0x1FFFFF)
    t_b = jnp.where(t_s < 0, MININT - t_s, t_s)
    thr_ref[...] = lax.bitcast_convert_type(t_b, jnp.float32)[:, 0]


def _phase1(xy_half, xyt):
    return pl.pallas_call(
        _phase1_body,
        grid=(NH // BQ,),
        in_specs=[
            pl.BlockSpec((BQ, 2), lambda i: (i, 0)),
            pl.BlockSpec((2, N), lambda i: (0, 0)),
        ],
        out_specs=[
            pl.BlockSpec((BQ, N), lambda i: (i, 0)),
            pl.BlockSpec((BQ,), lambda i: (i,)),
        ],
        out_shape=[
            jax.ShapeDtypeStruct((NH, N), jnp.float32),
            jax.ShapeDtypeStruct((NH,), jnp.float32),
        ],
    )(xy_half, xyt)


def _merge_runs(ak, av, bk, bv):
    m = len(ak)
    sk = list(ak) + [lax.rev(k, (0,)) for k in reversed(bk)]
    sv = list(av) + [lax.rev(v, (0,)) for v in reversed(bv)]
    total = 2 * m
    stride = m
    while stride >= 1:
        for blk in range(0, total, 2 * stride):
            for i in range(blk, blk + stride):
                ka, kb = sk[i], sk[i + stride]
                va, vb = sv[i], sv[i + stride]
                le = ka <= kb
                sk[i] = jnp.where(le, ka, kb)
                sk[i + stride] = jnp.where(le, kb, ka)
                sv[i] = jnp.where(le, va, vb)
                sv[i + stride] = jnp.where(le, vb, va)
        stride //= 2
    for i in range(total):
        sk[i], sv[i] = plsc.sort_key_val(sk[i], sv[i])
    return sk, sv


def _merge_lower_half(ak, av, bk, bv):
    m = len(ak)
    sk = list(ak) + [lax.rev(k, (0,)) for k in reversed(bk)]
    sv = list(av) + [lax.rev(v, (0,)) for v in reversed(bv)]
    lk, lv = [], []
    for i in range(m):
        le = sk[i] <= sk[i + m]
        lk.append(jnp.where(le, sk[i], sk[i + m]))
        lv.append(jnp.where(le, sv[i], sv[i + m]))
    stride = m // 2
    while stride >= 1:
        for blk in range(0, m, 2 * stride):
            for i in range(blk, blk + stride):
                ka, kb = lk[i], lk[i + stride]
                va, vb = lv[i], lv[i + stride]
                le = ka <= kb
                lk[i] = jnp.where(le, ka, kb)
                lk[i + stride] = jnp.where(le, kb, ka)
                lv[i] = jnp.where(le, va, vb)
                lv[i + stride] = jnp.where(le, vb, va)
        stride //= 2
    for i in range(m):
        lk[i], lv[i] = plsc.sort_key_val(lk[i], lv[i])
    return lk, lv


def _sort_candidates(cd, ci):
    nv = CAP // 16
    ks, vs = [], []
    for i in range(nv):
        k = cd[pl.ds(i * 16, 16)]
        v = ci[pl.ds(i * 16, 16)]
        k, v = plsc.sort_key_val(k, v)
        ks.append([k])
        vs.append([v])
    while len(ks) > 2:
        nk, nvv = [], []
        for i in range(0, len(ks), 2):
            mk, mv = _merge_runs(ks[i], vs[i], ks[i + 1], vs[i + 1])
            nk.append(mk)
            nvv.append(mv)
        ks, vs = nk, nvv
    return _merge_lower_half(ks[0], vs[0], ks[1], vs[1])


def _sc_make_kernel():
    mesh = plsc.VectorSubcoreMesh(core_axis_name="c", subcore_axis_name="s")

    @functools.partial(
        pl.kernel,
        mesh=mesh,
        compiler_params=pltpu.CompilerParams(needs_layout_passes=False),
        out_type=jax.ShapeDtypeStruct((NH, 2 * OUTW), jnp.float32),
        scratch_types=[
            pltpu.VMEM((N,), jnp.float32),
            pltpu.VMEM((N,), jnp.float32),
            pltpu.VMEM((RPW + 16,), jnp.float32),
            pltpu.VMEM((CAP,), jnp.float32),
            pltpu.VMEM((CAP,), jnp.int32),
            pltpu.VMEM((16,), jnp.int32),
            pltpu.VMEM((OUTW + 32,), jnp.float32),
            pltpu.VMEM((OUTW + 32,), jnp.int32),
            pltpu.VMEM((2 * OUTW,), jnp.float32),
            pltpu.VMEM((2 * OUTW,), jnp.float32),
            pltpu.SemaphoreType.DMA,
            pltpu.SemaphoreType.DMA,
            pltpu.SemaphoreType.DMA,
            pltpu.SemaphoreType.DMA,
            pltpu.SemaphoreType.DMA,
        ],
    )
    def sc_select(d2_hbm, thr_hbm, outf_hbm,
                  rowbuf0, rowbuf1, thrbuf, cd, ci, posref, od, oi,
                  ob0, ob1, sem0, sem1, semt, osem0, osem1):
        cid = lax.axis_index("c")
        sid = lax.axis_index("s")
        wid = cid * 16 + sid
        base = wid * RPW
        ii = lax.iota(jnp.int32, 16)
        inf = jnp.full((16,), float("inf"), jnp.float32)
        big = jnp.full((16,), N, jnp.int32)

        thr_dst = thrbuf.at[pl.ds(0, RPW)]
        pltpu.make_async_copy(thr_hbm.at[pl.ds(base, RPW)], thr_dst, semt).start()
        pltpu.make_async_copy(thr_hbm.at[pl.ds(base, RPW)], thr_dst, semt).wait()

        def start_in(row, buf, sem):
            pltpu.make_async_copy(d2_hbm.at[row], buf, sem).start()

        def wait_in(row, buf, sem):
            pltpu.make_async_copy(d2_hbm.at[row], buf, sem).wait()

        start_in(base, rowbuf0, sem0)
        start_in(base + 1, rowbuf1, sem1)

        def process(r, buf, ob, osem):
            row = base + r

            @pl.when(r >= 2)
            def _():
                pltpu.make_async_copy(ob, outf_hbm.at[row], osem).wait()

            tv = jnp.full((16,), thrbuf[pl.ds(r, 16)][0])
            for i in range(CAP // 16):
                cd[pl.ds(i * 16, 16)] = inf
                ci[pl.ds(i * 16, 16)] = big
            posref[...] = jnp.zeros((16,), jnp.int32)

            def chunk(g, carry):
                b0 = g * 128
                vals = [buf[pl.ds(b0 + 16 * j, 16)] for j in range(8)]
                masks = [v <= tv for v in vals]
                o1 = (masks[0] | masks[1]) | (masks[2] | masks[3])
                o2 = (masks[4] | masks[5]) | (masks[6] | masks[7])
                anym = o1 | o2

                def collect(lo):
                    pos = posref[...]
                    for j in range(lo, lo + 4):
                        mj = masks[j]
                        cum = plsc.cumsum(jnp.where(mj, 1, 0).astype(jnp.int32))
                        wi = jnp.minimum(pos + cum - 1, CAP - 1)
                        plsc.store_scatter(cd, [wi], vals[j], mask=mj)
                        plsc.store_scatter(ci, [wi], ii + (b0 + 16 * j), mask=mj)
                        pos = pos + plsc.all_reduce_population_count(mj)
                    posref[...] = pos

                @pl.when(plsc.all_reduce_population_count(anym)[0] > 0)
                def _():
                    @pl.when(plsc.all_reduce_population_count(o1)[0] > 0)
                    def _():
                        collect(0)

                    @pl.when(plsc.all_reduce_population_count(o2)[0] > 0)
                    def _():
                        collect(4)

                return carry

            lax.fori_loop(0, N // 128, chunk, 0)

            ks, vs = _sort_candidates(cd, ci)
            od[pl.ds(0, 16)] = jnp.full((16,), float("-inf"), jnp.float32)
            oi[pl.ds(0, 16)] = jnp.zeros((16,), jnp.int32)
            od[pl.ds(OUTW + 16, 16)] = inf
            oi[pl.ds(OUTW + 16, 16)] = big
            for i in range(OUTW // 16):
                od[pl.ds(16 + i * 16, 16)] = ks[i]
                oi[pl.ds(16 + i * 16, 16)] = vs[i]
            even = (ii & 1) == 0
            for p in range(5):
                act = even if (p & 1) == 0 else ~even
                nxt = ~act
                newvs = []
                for i in range(OUTW // 16):
                    b = 16 + i * 16
                    k = od[pl.ds(b, 16)]
                    kp = od[pl.ds(b - 1, 16)]
                    kn = od[pl.ds(b + 1, 16)]
                    v = oi[pl.ds(b, 16)]
                    vp = oi[pl.ds(b - 1, 16)]
                    vn = oi[pl.ds(b + 1, 16)]
                    swap = act & (k == kn) & (v > vn)
                    swapp = nxt & (kp == k) & (vp > v)
                    newvs.append(jnp.where(swapp, vp, jnp.where(swap, vn, v)))
                for i in range(OUTW // 16):
                    oi[pl.ds(16 + i * 16, 16)] = newvs[i]
            for i in range(OUTW // 16):
                ob[pl.ds(i * 16, 16)] = od[pl.ds(16 + i * 16, 16)]
                ob[pl.ds(OUTW + i * 16, 16)] = plsc.bitcast(
                    oi[pl.ds(16 + i * 16, 16)], jnp.float32)
            pltpu.make_async_copy(ob, outf_hbm.at[row], osem).start()

        def rowpair(g, carry):
            r0 = 2 * g
            wait_in(base + r0, rowbuf0, sem0)
            process(r0, rowbuf0, ob0, osem0)

            @pl.when(r0 + 2 < RPW)
            def _():
                start_in(base + r0 + 2, rowbuf0, sem0)

            r1 = 2 * g + 1
            wait_in(base + r1, rowbuf1, sem1)
            process(r1, rowbuf1, ob1, osem1)

            @pl.when(r1 + 2 < RPW)
            def _():
                start_in(base + r1 + 2, rowbuf1, sem1)

            return carry

        lax.fori_loop(0, RPW // 2, rowpair, 0)
        pltpu.make_async_copy(ob0, outf_hbm.at[base], osem0).wait()
        pltpu.make_async_copy(ob1, outf_hbm.at[base], osem1).wait()

    return sc_select


_sc_selects = [_sc_make_kernel() for _ in range(H)]


def _score_body(nnd2_ref, score_ref, weight_ref, wr_ref):
    nn = nnd2_ref[...]
    nearest = jnp.sqrt(jnp.maximum(nn, 0.0) + 1e-12)
    distance = jnp.mean(nearest, axis=1, keepdims=True)

    b = lax.bitcast_convert_type(distance, jnp.int32)
    s = jnp.where(b < 0, MININT - b, b)
    kth = float(N - PCT_IDX)

    def step(i, low):
        bit = lax.shift_left(jnp.int32(1), jnp.int32(31) - i)
        cand = low + bit
        cnt = jnp.sum((s < cand).astype(jnp.float32))
        return jnp.where(cnt < kth, cand, low)

    low = lax.fori_loop(0, 32, step, jnp.int32(MININT))
    t_b = jnp.where(low < 0, MININT - low, low)
    cutoff = lax.bitcast_convert_type(t_b, jnp.float32)

    ds = distance / cutoff
    ds = jnp.where(ds >= 1.0, 1.0, ds)
    score_ref[...] = (1.0 - ds)[:, 0]

    w = nearest / cutoff
    w = jnp.where(w >= 1.0, 1.0, w)
    weight = 1.0 - w
    weight_ref[...] = weight
    wsum = jnp.sum(weight)
    wcnt = jnp.sum((weight > -1.0).astype(jnp.float32))
    wr_ref[...] = jnp.reshape(wsum / wcnt, (1, 1))


def _score(nn_d2):
    return pl.pallas_call(
        _score_body,
        out_shape=[
            jax.ShapeDtypeStruct((N,), jnp.float32),
            jax.ShapeDtypeStruct((N, K), jnp.float32),
            jax.ShapeDtypeStruct((1, 1), jnp.float32),
        ],
    )(nn_d2)


def kernel(xy_index):
    xyt = xy_index.T
    parts = []
    for h in range(H):
        xy_half = lax.slice(xy_index, (h * NH, 0), ((h + 1) * NH, 2))
        d2_h, thr_h = _phase1(xy_half, xyt)
        parts.append(_sc_selects[h](d2_h, thr_h))
    outf = jnp.concatenate(parts, axis=0)
    sample_neighbor = lax.bitcast_convert_type(
        outf[:, OUTW + 1:OUTW + KSEL], jnp.int32)
    nn_d2 = outf[:, 1:KSEL]
    distance_score, sample_neighbor_weight, wr = _score(nn_d2)
    return (distance_score, sample_neighbor_weight, sample_neighbor,
            wr.reshape(()))


# speedup vs baseline: 1.0695x; 1.0695x over previous
import functools

import jax
import jax.numpy as jnp
from jax import lax
from jax.experimental import pallas as pl
from jax.experimental.pallas import tpu as pltpu
from jax.experimental.pallas import tpu_sc as plsc

N = 16384
K = 100
KSEL = K + 1
PCT_IDX = 16
BQ = 128
NW = 32
H = 1
NH = N // H
RPW = NH // NW
CAP = 256
OUTW = 128
MININT = -2147483648


def _phase1_body(xy_ref, xyt_ref, d2_ref, thr_ref):
    q = xy_ref[...]
    kxy = xyt_ref[...]
    qx = q[:, 0:1]
    qy = q[:, 1:2]
    kx = kxy[0:1, :]
    ky = kxy[1:2, :]
    qsq = qx * qx + qy * qy
    ksq = kx * kx + ky * ky
    dot = lax.dot_general(q, kxy, (((1,), (0,)), ((), ())),
                          preferred_element_type=jnp.float32)
    d2 = qsq - 2.0 * dot + ksq
    d2_ref[...] = d2

    b = lax.bitcast_convert_type(d2, jnp.int32)
    s = jnp.where(b < 0, MININT - b, b)

    def step(i, low):
        bit = lax.shift_left(jnp.int32(1), jnp.int32(31) - i)
        cand = low + bit
        cnt = jnp.sum((s < cand).astype(jnp.float32), axis=1, keepdims=True)
        return jnp.where(cnt < float(KSEL), cand, low)

    low = lax.fori_loop(0, 13, step, jnp.full((BQ, 1), MININT, jnp.int32))
    t_s = low + jnp.int32(0x7FFFF)
    t_b = jnp.where(t_s < 0, MININT - t_s, t_s)
    thr_ref[...] = lax.bitcast_convert_type(t_b, jnp.float32)[:, 0]


def _phase1(xy_half, xyt):
    return pl.pallas_call(
        _phase1_body,
        grid=(NH // BQ,),
        in_specs=[
            pl.BlockSpec((BQ, 2), lambda i: (i, 0)),
            pl.BlockSpec((2, N), lambda i: (0, 0)),
        ],
        out_specs=[
            pl.BlockSpec((BQ, N), lambda i: (i, 0)),
            pl.BlockSpec((BQ,), lambda i: (i,)),
        ],
        out_shape=[
            jax.ShapeDtypeStruct((NH, N), jnp.float32),
            jax.ShapeDtypeStruct((NH,), jnp.float32),
        ],
    )(xy_half, xyt)


def _merge_runs(ak, av, bk, bv):
    m = len(ak)
    sk = list(ak) + [lax.rev(k, (0,)) for k in reversed(bk)]
    sv = list(av) + [lax.rev(v, (0,)) for v in reversed(bv)]
    total = 2 * m
    stride = m
    while stride >= 1:
        for blk in range(0, total, 2 * stride):
            for i in range(blk, blk + stride):
                ka, kb = sk[i], sk[i + stride]
                va, vb = sv[i], sv[i + stride]
                le = ka <= kb
                sk[i] = jnp.where(le, ka, kb)
                sk[i + stride] = jnp.where(le, kb, ka)
                sv[i] = jnp.where(le, va, vb)
                sv[i + stride] = jnp.where(le, vb, va)
        stride //= 2
    for i in range(total):
        sk[i], sv[i] = plsc.sort_key_val(sk[i], sv[i])
    return sk, sv


def _merge_lower_half(ak, av, bk, bv):
    m = len(ak)
    sk = list(ak) + [lax.rev(k, (0,)) for k in reversed(bk)]
    sv = list(av) + [lax.rev(v, (0,)) for v in reversed(bv)]
    lk, lv = [], []
    for i in range(m):
        le = sk[i] <= sk[i + m]
        lk.append(jnp.where(le, sk[i], sk[i + m]))
        lv.append(jnp.where(le, sv[i], sv[i + m]))
    stride = m // 2
    while stride >= 1:
        for blk in range(0, m, 2 * stride):
            for i in range(blk, blk + stride):
                ka, kb = lk[i], lk[i + stride]
                va, vb = lv[i], lv[i + stride]
                le = ka <= kb
                lk[i] = jnp.where(le, ka, kb)
                lk[i + stride] = jnp.where(le, kb, ka)
                lv[i] = jnp.where(le, va, vb)
                lv[i + stride] = jnp.where(le, vb, va)
        stride //= 2
    for i in range(m):
        lk[i], lv[i] = plsc.sort_key_val(lk[i], lv[i])
    return lk, lv


def _sort_candidates(cd, ci):
    nv = CAP // 16
    ks, vs = [], []
    for i in range(nv):
        k = cd[pl.ds(i * 16, 16)]
        v = ci[pl.ds(i * 16, 16)]
        k, v = plsc.sort_key_val(k, v)
        ks.append([k])
        vs.append([v])
    while len(ks) > 2:
        nk, nvv = [], []
        for i in range(0, len(ks), 2):
            mk, mv = _merge_runs(ks[i], vs[i], ks[i + 1], vs[i + 1])
            nk.append(mk)
            nvv.append(mv)
        ks, vs = nk, nvv
    return _merge_lower_half(ks[0], vs[0], ks[1], vs[1])


def _sc_make_kernel():
    mesh = plsc.VectorSubcoreMesh(core_axis_name="c", subcore_axis_name="s")

    @functools.partial(
        pl.kernel,
        mesh=mesh,
        compiler_params=pltpu.CompilerParams(needs_layout_passes=False),
        out_type=jax.ShapeDtypeStruct((NH, 2 * OUTW), jnp.float32),
        scratch_types=[
            pltpu.VMEM((N,), jnp.float32),
            pltpu.VMEM((N,), jnp.float32),
            pltpu.VMEM((RPW + 16,), jnp.float32),
            pltpu.VMEM((CAP,), jnp.float32),
            pltpu.VMEM((CAP,), jnp.int32),
            pltpu.VMEM((16,), jnp.int32),
            pltpu.VMEM((OUTW + 32,), jnp.float32),
            pltpu.VMEM((OUTW + 32,), jnp.int32),
            pltpu.VMEM((2 * OUTW,), jnp.float32),
            pltpu.VMEM((2 * OUTW,), jnp.float32),
            pltpu.SemaphoreType.DMA,
            pltpu.SemaphoreType.DMA,
            pltpu.SemaphoreType.DMA,
            pltpu.SemaphoreType.DMA,
            pltpu.SemaphoreType.DMA,
        ],
    )
    def sc_select(d2_hbm, thr_hbm, outf_hbm,
                  rowbuf0, rowbuf1, thrbuf, cd, ci, posref, od, oi,
                  ob0, ob1, sem0, sem1, semt, osem0, osem1):
        cid = lax.axis_index("c")
        sid = lax.axis_index("s")
        wid = cid * 16 + sid
        base = wid * RPW
        ii = lax.iota(jnp.int32, 16)
        inf = jnp.full((16,), float("inf"), jnp.float32)
        big = jnp.full((16,), N, jnp.int32)

        thr_dst = thrbuf.at[pl.ds(0, RPW)]
        pltpu.make_async_copy(thr_hbm.at[pl.ds(base, RPW)], thr_dst, semt).start()
        pltpu.make_async_copy(thr_hbm.at[pl.ds(base, RPW)], thr_dst, semt).wait()

        def start_in(row, buf, sem):
            pltpu.make_async_copy(d2_hbm.at[row], buf, sem).start()

        def wait_in(row, buf, sem):
            pltpu.make_async_copy(d2_hbm.at[row], buf, sem).wait()

        start_in(base, rowbuf0, sem0)
        start_in(base + 1, rowbuf1, sem1)

        def process(r, buf, ob, osem):
            row = base + r

            @pl.when(r >= 2)
            def _():
                pltpu.make_async_copy(ob, outf_hbm.at[row], osem).wait()

            tv = jnp.full((16,), thrbuf[pl.ds(r, 16)][0])
            for i in range(CAP // 16):
                cd[pl.ds(i * 16, 16)] = inf
                ci[pl.ds(i * 16, 16)] = big
            posref[...] = jnp.zeros((16,), jnp.int32)

            def chunk(g, carry):
                b0 = g * 128
                vals = [buf[pl.ds(b0 + 16 * j, 16)] for j in range(8)]
                masks = [v <= tv for v in vals]
                o1 = (masks[0] | masks[1]) | (masks[2] | masks[3])
                o2 = (masks[4] | masks[5]) | (masks[6] | masks[7])
                anym = o1 | o2

                @pl.when(plsc.all_reduce_population_count(anym)[0] > 0)
                def _():
                    pos = posref[...]
                    for j in range(8):
                        mj = masks[j]
                        cum = plsc.cumsum(jnp.where(mj, 1, 0).astype(jnp.int32))
                        wi = jnp.minimum(pos + cum - 1, CAP - 1)
                        plsc.store_scatter(cd, [wi], vals[j], mask=mj)
                        plsc.store_scatter(ci, [wi], ii + (b0 + 16 * j), mask=mj)
                        pos = pos + plsc.all_reduce_population_count(mj)
                    posref[...] = pos

                return carry

            lax.fori_loop(0, N // 128, chunk, 0)

            ks, vs = _sort_candidates(cd, ci)
            od[pl.ds(0, 16)] = jnp.full((16,), float("-inf"), jnp.float32)
            oi[pl.ds(0, 16)] = jnp.zeros((16,), jnp.int32)
            od[pl.ds(OUTW + 16, 16)] = inf
            oi[pl.ds(OUTW + 16, 16)] = big
            for i in range(OUTW // 16):
                od[pl.ds(16 + i * 16, 16)] = ks[i]
                oi[pl.ds(16 + i * 16, 16)] = vs[i]
            even = (ii & 1) == 0
            for p in range(5):
                act = even if (p & 1) == 0 else ~even
                nxt = ~act
                newvs = []
                for i in range(OUTW // 16):
                    b = 16 + i * 16
                    k = od[pl.ds(b, 16)]
                    kp = od[pl.ds(b - 1, 16)]
                    kn = od[pl.ds(b + 1, 16)]
                    v = oi[pl.ds(b, 16)]
                    vp = oi[pl.ds(b - 1, 16)]
                    vn = oi[pl.ds(b + 1, 16)]
                    swap = act & (k == kn) & (v > vn)
                    swapp = nxt & (kp == k) & (vp > v)
                    newvs.append(jnp.where(swapp, vp, jnp.where(swap, vn, v)))
                for i in range(OUTW // 16):
                    oi[pl.ds(16 + i * 16, 16)] = newvs[i]
            for i in range(OUTW // 16):
                ob[pl.ds(i * 16, 16)] = od[pl.ds(16 + i * 16, 16)]
                ob[pl.ds(OUTW + i * 16, 16)] = plsc.bitcast(
                    oi[pl.ds(16 + i * 16, 16)], jnp.float32)
            pltpu.make_async_copy(ob, outf_hbm.at[row], osem).start()

        def rowpair(g, carry):
            r0 = 2 * g
            wait_in(base + r0, rowbuf0, sem0)
            process(r0, rowbuf0, ob0, osem0)

            @pl.when(r0 + 2 < RPW)
            def _():
                start_in(base + r0 + 2, rowbuf0, sem0)

            r1 = 2 * g + 1
            wait_in(base + r1, rowbuf1, sem1)
            process(r1, rowbuf1, ob1, osem1)

            @pl.when(r1 + 2 < RPW)
            def _():
                start_in(base + r1 + 2, rowbuf1, sem1)

            return carry

        lax.fori_loop(0, RPW // 2, rowpair, 0)
        pltpu.make_async_copy(ob0, outf_hbm.at[base], osem0).wait()
        pltpu.make_async_copy(ob1, outf_hbm.at[base], osem1).wait()

    return sc_select


_sc_selects = [_sc_make_kernel() for _ in range(H)]


def _score_body(nnd2_ref, score_ref, weight_ref, wr_ref):
    nn = nnd2_ref[...]
    nearest = jnp.sqrt(jnp.maximum(nn, 0.0) + 1e-12)
    distance = jnp.mean(nearest, axis=1, keepdims=True)

    b = lax.bitcast_convert_type(distance, jnp.int32)
    s = jnp.where(b < 0, MININT - b, b)
    kth = float(N - PCT_IDX)

    def step(i, low):
        bit = lax.shift_left(jnp.int32(1), jnp.int32(31) - i)
        cand = low + bit
        cnt = jnp.sum((s < cand).astype(jnp.float32))
        return jnp.where(cnt < kth, cand, low)

    low = lax.fori_loop(0, 32, step, jnp.int32(MININT))
    t_b = jnp.where(low < 0, MININT - low, low)
    cutoff = lax.bitcast_convert_type(t_b, jnp.float32)

    ds = distance / cutoff
    ds = jnp.where(ds >= 1.0, 1.0, ds)
    score_ref[...] = (1.0 - ds)[:, 0]

    w = nearest / cutoff
    w = jnp.where(w >= 1.0, 1.0, w)
    weight = 1.0 - w
    weight_ref[...] = weight
    wsum = jnp.sum(weight)
    wcnt = jnp.sum((weight > -1.0).astype(jnp.float32))
    wr_ref[...] = jnp.reshape(wsum / wcnt, (1, 1))


def _score(nn_d2):
    return pl.pallas_call(
        _score_body,
        out_shape=[
            jax.ShapeDtypeStruct((N,), jnp.float32),
            jax.ShapeDtypeStruct((N, K), jnp.float32),
            jax.ShapeDtypeStruct((1, 1), jnp.float32),
        ],
    )(nn_d2)


def kernel(xy_index):
    xyt = xy_index.T
    parts = []
    for h in range(H):
        xy_half = lax.slice(xy_index, (h * NH, 0), ((h + 1) * NH, 2))
        d2_h, thr_h = _phase1(xy_half, xyt)
        parts.append(_sc_selects[h](d2_h, thr_h))
    outf = jnp.concatenate(parts, axis=0)
    sample_neighbor = lax.bitcast_convert_type(
        outf[:, OUTW + 1:OUTW + KSEL], jnp.int32)
    nn_d2 = outf[:, 1:KSEL]
    distance_score, sample_neighbor_weight, wr = _score(nn_d2)
    return (distance_score, sample_neighbor_weight, sample_neighbor,
            wr.reshape(()))


# speedup vs baseline: 1.1053x; 1.0334x over previous
import functools

import jax
import jax.numpy as jnp
from jax import lax
from jax.experimental import pallas as pl
from jax.experimental.pallas import tpu as pltpu
from jax.experimental.pallas import tpu_sc as plsc

N = 16384
K = 100
KSEL = K + 1
PCT_IDX = 16
BQ = 256
NW = 32
H = 1
NH = N // H
RPW = NH // NW
CAP = 256
OUTW = 128
MININT = -2147483648


def _phase1_body(xy_ref, xyt_ref, d2_ref, thr_ref):
    q = xy_ref[...]
    kxy = xyt_ref[...]
    qx = q[:, 0:1]
    qy = q[:, 1:2]
    kx = kxy[0:1, :]
    ky = kxy[1:2, :]
    qsq = qx * qx + qy * qy
    ksq = kx * kx + ky * ky
    dot = lax.dot_general(q, kxy, (((1,), (0,)), ((), ())),
                          preferred_element_type=jnp.float32)
    d2 = qsq - 2.0 * dot + ksq
    d2_ref[...] = d2

    b = lax.bitcast_convert_type(d2, jnp.int32)
    s = jnp.where(b < 0, MININT - b, b)

    def step(i, low):
        bit = lax.shift_left(jnp.int32(1), jnp.int32(31) - i)
        cand = low + bit
        cnt = jnp.sum((s < cand).astype(jnp.float32), axis=1, keepdims=True)
        return jnp.where(cnt < float(KSEL), cand, low)

    low = lax.fori_loop(0, 13, step, jnp.full((BQ, 1), MININT, jnp.int32))
    t_s = low + jnp.int32(0x7FFFF)
    t_b = jnp.where(t_s < 0, MININT - t_s, t_s)
    thr_ref[...] = lax.bitcast_convert_type(t_b, jnp.float32)[:, 0]


def _phase1(xy_half, xyt):
    return pl.pallas_call(
        _phase1_body,
        grid=(NH // BQ,),
        in_specs=[
            pl.BlockSpec((BQ, 2), lambda i: (i, 0)),
            pl.BlockSpec((2, N), lambda i: (0, 0)),
        ],
        out_specs=[
            pl.BlockSpec((BQ, N), lambda i: (i, 0)),
            pl.BlockSpec((BQ,), lambda i: (i,)),
        ],
        out_shape=[
            jax.ShapeDtypeStruct((NH, N), jnp.float32),
            jax.ShapeDtypeStruct((NH,), jnp.float32),
        ],
    )(xy_half, xyt)


def _merge_runs(ak, av, bk, bv):
    m = len(ak)
    sk = list(ak) + [lax.rev(k, (0,)) for k in reversed(bk)]
    sv = list(av) + [lax.rev(v, (0,)) for v in reversed(bv)]
    total = 2 * m
    stride = m
    while stride >= 1:
        for blk in range(0, total, 2 * stride):
            for i in range(blk, blk + stride):
                ka, kb = sk[i], sk[i + stride]
                va, vb = sv[i], sv[i + stride]
                le = ka <= kb
                sk[i] = jnp.where(le, ka, kb)
                sk[i + stride] = jnp.where(le, kb, ka)
                sv[i] = jnp.where(le, va, vb)
                sv[i + stride] = jnp.where(le, vb, va)
        stride //= 2
    for i in range(total):
        sk[i], sv[i] = plsc.sort_key_val(sk[i], sv[i])
    return sk, sv


def _merge_lower_half(ak, av, bk, bv):
    m = len(ak)
    sk = list(ak) + [lax.rev(k, (0,)) for k in reversed(bk)]
    sv = list(av) + [lax.rev(v, (0,)) for v in reversed(bv)]
    lk, lv = [], []
    for i in range(m):
        le = sk[i] <= sk[i + m]
        lk.append(jnp.where(le, sk[i], sk[i + m]))
        lv.append(jnp.where(le, sv[i], sv[i + m]))
    stride = m // 2
    while stride >= 1:
        for blk in range(0, m, 2 * stride):
            for i in range(blk, blk + stride):
                ka, kb = lk[i], lk[i + stride]
                va, vb = lv[i], lv[i + stride]
                le = ka <= kb
                lk[i] = jnp.where(le, ka, kb)
                lk[i + stride] = jnp.where(le, kb, ka)
                lv[i] = jnp.where(le, va, vb)
                lv[i + stride] = jnp.where(le, vb, va)
        stride //= 2
    for i in range(m):
        lk[i], lv[i] = plsc.sort_key_val(lk[i], lv[i])
    return lk, lv


def _sort_candidates(cd, ci):
    nv = CAP // 16
    ks, vs = [], []
    for i in range(nv):
        k = cd[pl.ds(i * 16, 16)]
        v = ci[pl.ds(i * 16, 16)]
        k, v = plsc.sort_key_val(k, v)
        ks.append([k])
        vs.append([v])
    while len(ks) > 2:
        nk, nvv = [], []
        for i in range(0, len(ks), 2):
            mk, mv = _merge_runs(ks[i], vs[i], ks[i + 1], vs[i + 1])
            nk.append(mk)
            nvv.append(mv)
        ks, vs = nk, nvv
    return _merge_lower_half(ks[0], vs[0], ks[1], vs[1])


def _sc_make_kernel():
    mesh = plsc.VectorSubcoreMesh(core_axis_name="c", subcore_axis_name="s")

    @functools.partial(
        pl.kernel,
        mesh=mesh,
        compiler_params=pltpu.CompilerParams(needs_layout_passes=False),
        out_type=jax.ShapeDtypeStruct((NH, 2 * OUTW), jnp.float32),
        scratch_types=[
            pltpu.VMEM((N,), jnp.float32),
            pltpu.VMEM((N,), jnp.float32),
            pltpu.VMEM((RPW + 16,), jnp.float32),
            pltpu.VMEM((CAP,), jnp.float32),
            pltpu.VMEM((CAP,), jnp.int32),
            pltpu.VMEM((16,), jnp.int32),
            pltpu.VMEM((OUTW + 32,), jnp.float32),
            pltpu.VMEM((OUTW + 32,), jnp.int32),
            pltpu.VMEM((2 * OUTW,), jnp.float32),
            pltpu.VMEM((2 * OUTW,), jnp.float32),
            pltpu.SemaphoreType.DMA,
            pltpu.SemaphoreType.DMA,
            pltpu.SemaphoreType.DMA,
            pltpu.SemaphoreType.DMA,
            pltpu.SemaphoreType.DMA,
        ],
    )
    def sc_select(d2_hbm, thr_hbm, outf_hbm,
                  rowbuf0, rowbuf1, thrbuf, cd, ci, posref, od, oi,
                  ob0, ob1, sem0, sem1, semt, osem0, osem1):
        cid = lax.axis_index("c")
        sid = lax.axis_index("s")
        wid = cid * 16 + sid
        base = wid * RPW
        ii = lax.iota(jnp.int32, 16)
        inf = jnp.full((16,), float("inf"), jnp.float32)
        big = jnp.full((16,), N, jnp.int32)

        thr_dst = thrbuf.at[pl.ds(0, RPW)]
        pltpu.make_async_copy(thr_hbm.at[pl.ds(base, RPW)], thr_dst, semt).start()
        pltpu.make_async_copy(thr_hbm.at[pl.ds(base, RPW)], thr_dst, semt).wait()

        def start_in(row, buf, sem):
            pltpu.make_async_copy(d2_hbm.at[row], buf, sem).start()

        def wait_in(row, buf, sem):
            pltpu.make_async_copy(d2_hbm.at[row], buf, sem).wait()

        start_in(base, rowbuf0, sem0)
        start_in(base + 1, rowbuf1, sem1)

        def process(r, buf, ob, osem):
            row = base + r

            @pl.when(r >= 2)
            def _():
                pltpu.make_async_copy(ob, outf_hbm.at[row], osem).wait()

            tv = jnp.full((16,), thrbuf[pl.ds(r, 16)][0])
            for i in range(CAP // 16):
                cd[pl.ds(i * 16, 16)] = inf
                ci[pl.ds(i * 16, 16)] = big
            posref[...] = jnp.zeros((16,), jnp.int32)

            def chunk(g, carry):
                b0 = g * 128
                vals = [buf[pl.ds(b0 + 16 * j, 16)] for j in range(8)]
                masks = [v <= tv for v in vals]
                o1 = (masks[0] | masks[1]) | (masks[2] | masks[3])
                o2 = (masks[4] | masks[5]) | (masks[6] | masks[7])
                anym = o1 | o2

                @pl.when(plsc.all_reduce_population_count(anym)[0] > 0)
                def _():
                    pos = posref[...]
                    for j in range(8):
                        mj = masks[j]
                        cum = plsc.cumsum(jnp.where(mj, 1, 0).astype(jnp.int32))
                        wi = jnp.minimum(pos + cum - 1, CAP - 1)
                        plsc.store_scatter(cd, [wi], vals[j], mask=mj)
                        plsc.store_scatter(ci, [wi], ii + (b0 + 16 * j), mask=mj)
                        pos = pos + plsc.all_reduce_population_count(mj)
                    posref[...] = pos

                return carry

            lax.fori_loop(0, N // 128, chunk, 0)

            ks, vs = _sort_candidates(cd, ci)
            od[pl.ds(0, 16)] = jnp.full((16,), float("-inf"), jnp.float32)
            oi[pl.ds(0, 16)] = jnp.zeros((16,), jnp.int32)
            od[pl.ds(OUTW + 16, 16)] = inf
            oi[pl.ds(OUTW + 16, 16)] = big
            for i in range(OUTW // 16):
                od[pl.ds(16 + i * 16, 16)] = ks[i]
                oi[pl.ds(16 + i * 16, 16)] = vs[i]
            even = (ii & 1) == 0
            for p in range(5):
                act = even if (p & 1) == 0 else ~even
                nxt = ~act
                newvs = []
                for i in range(OUTW // 16):
                    b = 16 + i * 16
                    k = od[pl.ds(b, 16)]
                    kp = od[pl.ds(b - 1, 16)]
                    kn = od[pl.ds(b + 1, 16)]
                    v = oi[pl.ds(b, 16)]
                    vp = oi[pl.ds(b - 1, 16)]
                    vn = oi[pl.ds(b + 1, 16)]
                    swap = act & (k == kn) & (v > vn)
                    swapp = nxt & (kp == k) & (vp > v)
                    newvs.append(jnp.where(swapp, vp, jnp.where(swap, vn, v)))
                for i in range(OUTW // 16):
                    oi[pl.ds(16 + i * 16, 16)] = newvs[i]
            for i in range(OUTW // 16):
                ob[pl.ds(i * 16, 16)] = od[pl.ds(16 + i * 16, 16)]
                ob[pl.ds(OUTW + i * 16, 16)] = plsc.bitcast(
                    oi[pl.ds(16 + i * 16, 16)], jnp.float32)
            pltpu.make_async_copy(ob, outf_hbm.at[row], osem).start()

        def rowpair(g, carry):
            r0 = 2 * g
            wait_in(base + r0, rowbuf0, sem0)
            process(r0, rowbuf0, ob0, osem0)

            @pl.when(r0 + 2 < RPW)
            def _():
                start_in(base + r0 + 2, rowbuf0, sem0)

            r1 = 2 * g + 1
            wait_in(base + r1, rowbuf1, sem1)
            process(r1, rowbuf1, ob1, osem1)

            @pl.when(r1 + 2 < RPW)
            def _():
                start_in(base + r1 + 2, rowbuf1, sem1)

            return carry

        lax.fori_loop(0, RPW // 2, rowpair, 0)
        pltpu.make_async_copy(ob0, outf_hbm.at[base], osem0).wait()
        pltpu.make_async_copy(ob1, outf_hbm.at[base], osem1).wait()

    return sc_select


_sc_selects = [_sc_make_kernel() for _ in range(H)]


def _score_body(nnd2_ref, score_ref, weight_ref, wr_ref):
    nn = nnd2_ref[...]
    nearest = jnp.sqrt(jnp.maximum(nn, 0.0) + 1e-12)
    distance = jnp.mean(nearest, axis=1, keepdims=True)

    b = lax.bitcast_convert_type(distance, jnp.int32)
    s = jnp.where(b < 0, MININT - b, b)
    kth = float(N - PCT_IDX)

    def step(i, low):
        bit = lax.shift_left(jnp.int32(1), jnp.int32(31) - i)
        cand = low + bit
        cnt = jnp.sum((s < cand).astype(jnp.float32))
        return jnp.where(cnt < kth, cand, low)

    low = lax.fori_loop(0, 32, step, jnp.int32(MININT))
    t_b = jnp.where(low < 0, MININT - low, low)
    cutoff = lax.bitcast_convert_type(t_b, jnp.float32)

    ds = distance / cutoff
    ds = jnp.where(ds >= 1.0, 1.0, ds)
    score_ref[...] = (1.0 - ds)[:, 0]

    w = nearest / cutoff
    w = jnp.where(w >= 1.0, 1.0, w)
    weight = 1.0 - w
    weight_ref[...] = weight
    wsum = jnp.sum(weight)
    wcnt = jnp.sum((weight > -1.0).astype(jnp.float32))
    wr_ref[...] = jnp.reshape(wsum / wcnt, (1, 1))


def _score(nn_d2):
    return pl.pallas_call(
        _score_body,
        out_shape=[
            jax.ShapeDtypeStruct((N,), jnp.float32),
            jax.ShapeDtypeStruct((N, K), jnp.float32),
            jax.ShapeDtypeStruct((1, 1), jnp.float32),
        ],
    )(nn_d2)


def kernel(xy_index):
    xyt = xy_index.T
    parts = []
    for h in range(H):
        xy_half = lax.slice(xy_index, (h * NH, 0), ((h + 1) * NH, 2))
        d2_h, thr_h = _phase1(xy_half, xyt)
        parts.append(_sc_selects[h](d2_h, thr_h))
    outf = jnp.concatenate(parts, axis=0)
    sample_neighbor = lax.bitcast_convert_type(
        outf[:, OUTW + 1:OUTW + KSEL], jnp.int32)
    nn_d2 = outf[:, 1:KSEL]
    distance_score, sample_neighbor_weight, wr = _score(nn_d2)
    return (distance_score, sample_neighbor_weight, sample_neighbor,
            wr.reshape(()))
